# Initial kernel scaffold; baseline (speedup 1.0000x reference)
import jax
import jax.numpy as jnp
from jax.experimental import pallas as pl


def kernel(predicted_continue_logits, rewards, continues, critic_values):
    raise NotImplementedError("write your pallas kernel here")


# speedup vs baseline: 2.3374x; 2.3374x over previous
import functools

import jax
import jax.numpy as jnp
from jax.experimental import pallas as pl
from jax.experimental.pallas import tpu as pltpu

_GAMMA = 0.997
_LAMDA = 0.95
_T = 64
_BLK = 1024


def _shift_left(x, d, fill):
    blk = x.shape[0]
    pad = jnp.full((blk, d), fill, dtype=x.dtype)
    return jnp.concatenate([x[:, d:], pad], axis=-1)


def _loss_kernel(logits_ref, rewards_ref, continues_ref, values_ref,
                 probs_ref, returns_ref):
    probs_ref[...] = jax.nn.sigmoid(logits_ref[...])

    r = rewards_ref[...]
    c = continues_ref[...]
    v = values_ref[...]

    vn = _shift_left(v, 1, 0.0)
    a = r + (_GAMMA * (1.0 - _LAMDA)) * c * vn
    b = (_GAMMA * _LAMDA) * c

    lane = jax.lax.broadcasted_iota(jnp.int32, a.shape, 1)
    last = lane >= _T - 1
    a = jnp.where(last, 0.0, a)
    b = jnp.where(last, 1.0, b)

    for d in (1, 2, 4, 8, 16, 32):
        a_s = _shift_left(a, d, 0.0)
        b_s = _shift_left(b, d, 1.0)
        a = a + b * a_s
        b = b * b_s

    bootstrap = v[:, _T - 1:_T]
    returns_ref[...] = (a + b * bootstrap)[:, :_T - 1]


@jax.jit
def kernel(predicted_continue_logits, rewards, continues, critic_values):
    B, T = predicted_continue_logits.shape
    grid = (B // _BLK,)
    in_spec = pl.BlockSpec((_BLK, T), lambda i: (i, 0))
    probs, returns = pl.pallas_call(
        _loss_kernel,
        grid=grid,
        in_specs=[in_spec, in_spec, in_spec, in_spec],
        out_specs=[
            pl.BlockSpec((_BLK, T), lambda i: (i, 0)),
            pl.BlockSpec((_BLK, T - 1), lambda i: (i, 0)),
        ],
        out_shape=[
            jax.ShapeDtypeStruct((B, T), jnp.float32),
            jax.ShapeDtypeStruct((B, T - 1), jnp.float32),
        ],
        compiler_params=pltpu.CompilerParams(
            dimension_semantics=("parallel",),
        ),
    )(predicted_continue_logits, rewards, continues, critic_values)
    return probs, returns


# speedup vs baseline: 2.3424x; 1.0021x over previous
import functools

import jax
import jax.numpy as jnp
from jax.experimental import pallas as pl
from jax.experimental.pallas import tpu as pltpu

_GAMMA = 0.997
_LAMDA = 0.95
_T = 64
_BLK = 4096


def _shift_left(x, d, fill):
    blk = x.shape[0]
    pad = jnp.full((blk, d), fill, dtype=x.dtype)
    return jnp.concatenate([x[:, d:], pad], axis=-1)


def _loss_kernel(logits_ref, rewards_ref, continues_ref, values_ref,
                 probs_ref, returns_ref):
    probs_ref[...] = jax.nn.sigmoid(logits_ref[...])

    r = rewards_ref[...]
    c = continues_ref[...]
    v = values_ref[...]

    vn = _shift_left(v, 1, 0.0)
    a = r + (_GAMMA * (1.0 - _LAMDA)) * c * vn
    b = (_GAMMA * _LAMDA) * c

    lane = jax.lax.broadcasted_iota(jnp.int32, a.shape, 1)
    last = lane >= _T - 1
    a = jnp.where(last, 0.0, a)
    b = jnp.where(last, 1.0, b)

    for d in (1, 2, 4, 8, 16, 32):
        a_s = _shift_left(a, d, 0.0)
        b_s = _shift_left(b, d, 1.0)
        a = a + b * a_s
        b = b * b_s

    bootstrap = v[:, _T - 1:_T]
    returns_ref[...] = (a + b * bootstrap)[:, :_T - 1]


@jax.jit
def kernel(predicted_continue_logits, rewards, continues, critic_values):
    B, T = predicted_continue_logits.shape
    grid = (B // _BLK,)
    in_spec = pl.BlockSpec((_BLK, T), lambda i: (i, 0))
    probs, returns = pl.pallas_call(
        _loss_kernel,
        grid=grid,
        in_specs=[in_spec, in_spec, in_spec, in_spec],
        out_specs=[
            pl.BlockSpec((_BLK, T), lambda i: (i, 0)),
            pl.BlockSpec((_BLK, T - 1), lambda i: (i, 0)),
        ],
        out_shape=[
            jax.ShapeDtypeStruct((B, T), jnp.float32),
            jax.ShapeDtypeStruct((B, T - 1), jnp.float32),
        ],
        compiler_params=pltpu.CompilerParams(
            dimension_semantics=("parallel",),
        ),
    )(predicted_continue_logits, rewards, continues, critic_values)
    return probs, returns


# speedup vs baseline: 9.7870x; 4.1783x over previous
import jax
import jax.numpy as jnp
from jax.experimental import pallas as pl
from jax.experimental.pallas import tpu as pltpu

_GAMMA = 0.997
_LAMDA = 0.95
_T = 64
_BLKB = 1024


def _shift_up(x, d, fill):
    pad = jnp.full((d, x.shape[1]), fill, dtype=x.dtype)
    return jnp.concatenate([x[d:, :], pad], axis=0)


def _loss_kernel(logits_ref, rewards_ref, continues_ref, values_ref,
                 probs_ref, returns_ref):
    probs_ref[...] = jax.nn.sigmoid(logits_ref[...])

    r = rewards_ref[...]
    c = continues_ref[...]
    v = values_ref[...]

    vn = _shift_up(v, 1, 0.0)
    a = r + (_GAMMA * (1.0 - _LAMDA)) * c * vn
    b = (_GAMMA * _LAMDA) * c

    row = jax.lax.broadcasted_iota(jnp.int32, a.shape, 0)
    last = row >= _T - 1
    a = jnp.where(last, 0.0, a)
    b = jnp.where(last, 1.0, b)

    for d in (1, 2, 4, 8, 16, 32):
        a_s = _shift_up(a, d, 0.0)
        b_s = _shift_up(b, d, 1.0)
        a = a + b * a_s
        b = b * b_s

    bootstrap = v[_T - 1:_T, :]
    returns_ref[...] = (a + b * bootstrap)[:_T - 1, :]


@jax.jit
def kernel(predicted_continue_logits, rewards, continues, critic_values):
    B, T = predicted_continue_logits.shape
    grid = (B // _BLKB,)
    in_spec = pl.BlockSpec((T, _BLKB), lambda i: (0, i))
    probs_t, returns_t = pl.pallas_call(
        _loss_kernel,
        grid=grid,
        in_specs=[in_spec, in_spec, in_spec, in_spec],
        out_specs=[
            pl.BlockSpec((T, _BLKB), lambda i: (0, i)),
            pl.BlockSpec((T - 1, _BLKB), lambda i: (0, i)),
        ],
        out_shape=[
            jax.ShapeDtypeStruct((T, B), jnp.float32),
            jax.ShapeDtypeStruct((T - 1, B), jnp.float32),
        ],
        compiler_params=pltpu.CompilerParams(
            dimension_semantics=("parallel",),
        ),
    )(
        predicted_continue_logits.T,
        rewards.T,
        continues.T,
        critic_values.T,
    )
    return probs_t.T, returns_t.T


# speedup vs baseline: 16.7427x; 1.7107x over previous
import jax
import jax.numpy as jnp
from jax.experimental import pallas as pl
from jax.experimental.pallas import tpu as pltpu

_GAMMA = 0.997
_LAMDA = 0.95
_T = 64
_BLKB = 4096


def _shift_up(x, d, fill):
    pad = jnp.full((d, x.shape[1]), fill, dtype=x.dtype)
    return jnp.concatenate([x[d:, :], pad], axis=0)


def _loss_kernel(logits_ref, rewards_ref, continues_ref, values_ref,
                 probs_ref, returns_ref):
    probs_ref[...] = jax.nn.sigmoid(logits_ref[...])

    r = rewards_ref[...]
    c = continues_ref[...]
    v = values_ref[...]

    vn = _shift_up(v, 1, 0.0)
    a = r + (_GAMMA * (1.0 - _LAMDA)) * c * vn
    b = (_GAMMA * _LAMDA) * c

    row = jax.lax.broadcasted_iota(jnp.int32, a.shape, 0)
    last = row >= _T - 1
    a = jnp.where(last, 0.0, a)
    b = jnp.where(last, 1.0, b)

    for d in (1, 2, 4, 8, 16, 32):
        a_s = _shift_up(a, d, 0.0)
        b_s = _shift_up(b, d, 1.0)
        a = a + b * a_s
        b = b * b_s

    bootstrap = v[_T - 1:_T, :]
    returns_ref[...] = (a + b * bootstrap)[:_T - 1, :]


@jax.jit
def kernel(predicted_continue_logits, rewards, continues, critic_values):
    B, T = predicted_continue_logits.shape
    grid = (B // _BLKB,)
    in_spec = pl.BlockSpec((T, _BLKB), lambda i: (0, i))
    probs_t, returns_t = pl.pallas_call(
        _loss_kernel,
        grid=grid,
        in_specs=[in_spec, in_spec, in_spec, in_spec],
        out_specs=[
            pl.BlockSpec((T, _BLKB), lambda i: (0, i)),
            pl.BlockSpec((T - 1, _BLKB), lambda i: (0, i)),
        ],
        out_shape=[
            jax.ShapeDtypeStruct((T, B), jnp.float32),
            jax.ShapeDtypeStruct((T - 1, B), jnp.float32),
        ],
        compiler_params=pltpu.CompilerParams(
            dimension_semantics=("parallel",),
        ),
    )(
        predicted_continue_logits.T,
        rewards.T,
        continues.T,
        critic_values.T,
    )
    return probs_t.T, returns_t.T


# speedup vs baseline: 18.1527x; 1.0842x over previous
import jax
import jax.numpy as jnp
from jax.experimental import pallas as pl
from jax.experimental.pallas import tpu as pltpu

_GAMMA = 0.997
_LAMDA = 0.95
_T = 64
_BLKB = 8192


def _shift_up(x, d, fill):
    pad = jnp.full((d, x.shape[1]), fill, dtype=x.dtype)
    return jnp.concatenate([x[d:, :], pad], axis=0)


def _loss_kernel(logits_ref, rewards_ref, continues_ref, values_ref,
                 probs_ref, returns_ref):
    probs_ref[...] = jax.nn.sigmoid(logits_ref[...])

    r = rewards_ref[...]
    c = continues_ref[...]
    v = values_ref[...]

    vn = _shift_up(v, 1, 0.0)
    a = r + (_GAMMA * (1.0 - _LAMDA)) * c * vn
    b = (_GAMMA * _LAMDA) * c

    row = jax.lax.broadcasted_iota(jnp.int32, a.shape, 0)
    last = row >= _T - 1
    a = jnp.where(last, 0.0, a)
    b = jnp.where(last, 1.0, b)

    for d in (1, 2, 4, 8, 16, 32):
        a_s = _shift_up(a, d, 0.0)
        b_s = _shift_up(b, d, 1.0)
        a = a + b * a_s
        b = b * b_s

    bootstrap = v[_T - 1:_T, :]
    returns_ref[...] = (a + b * bootstrap)[:_T - 1, :]


@jax.jit
def kernel(predicted_continue_logits, rewards, continues, critic_values):
    B, T = predicted_continue_logits.shape
    grid = (B // _BLKB,)
    in_spec = pl.BlockSpec((T, _BLKB), lambda i: (0, i))
    probs_t, returns_t = pl.pallas_call(
        _loss_kernel,
        grid=grid,
        in_specs=[in_spec, in_spec, in_spec, in_spec],
        out_specs=[
            pl.BlockSpec((T, _BLKB), lambda i: (0, i)),
            pl.BlockSpec((T - 1, _BLKB), lambda i: (0, i)),
        ],
        out_shape=[
            jax.ShapeDtypeStruct((T, B), jnp.float32),
            jax.ShapeDtypeStruct((T - 1, B), jnp.float32),
        ],
        compiler_params=pltpu.CompilerParams(
            dimension_semantics=("parallel",),
        ),
    )(
        predicted_continue_logits.T,
        rewards.T,
        continues.T,
        critic_values.T,
    )
    return probs_t.T, returns_t.T


# speedup vs baseline: 18.4304x; 1.0153x over previous
import jax
import jax.numpy as jnp
from jax.experimental import pallas as pl
from jax.experimental.pallas import tpu as pltpu

_GAMMA = 0.997
_LAMDA = 0.95
_T = 64
_BLKB = 8192


def _shift_up(x, d, fill):
    pad = jnp.full((d, x.shape[1]), fill, dtype=x.dtype)
    return jnp.concatenate([x[d:, :], pad], axis=0)


def _loss_kernel(logits_ref, rewards_ref, continues_ref, values_ref,
                 probs_ref, returns_ref):
    probs_ref[...] = jax.nn.sigmoid(logits_ref[...])

    r = rewards_ref[...]
    c = continues_ref[...]
    v = values_ref[...]

    vn = _shift_up(v, 1, 0.0)
    a = r + (_GAMMA * (1.0 - _LAMDA)) * c * vn
    b = (_GAMMA * _LAMDA) * c

    row = jax.lax.broadcasted_iota(jnp.int32, a.shape, 0)
    last = row >= _T - 1
    a = jnp.where(last, 0.0, a)
    b = jnp.where(last, 1.0, b)

    for d in (1, 2, 4, 8, 16):
        a_s = _shift_up(a, d, 0.0)
        b_s = _shift_up(b, d, 1.0)
        a = a + b * a_s
        b = b * b_s

    bootstrap = v[_T - 1:_T, :]
    a_s = _shift_up(a, 32, 0.0)
    b_s = _shift_up(b, 32, 1.0)
    returns_ref[...] = (a + b * (a_s + b_s * bootstrap))[:_T - 1, :]


@jax.jit
def kernel(predicted_continue_logits, rewards, continues, critic_values):
    B, T = predicted_continue_logits.shape
    grid = (B // _BLKB,)
    in_spec = pl.BlockSpec((T, _BLKB), lambda i: (0, i))
    probs_t, returns_t = pl.pallas_call(
        _loss_kernel,
        grid=grid,
        in_specs=[in_spec, in_spec, in_spec, in_spec],
        out_specs=[
            pl.BlockSpec((T, _BLKB), lambda i: (0, i)),
            pl.BlockSpec((T - 1, _BLKB), lambda i: (0, i)),
        ],
        out_shape=[
            jax.ShapeDtypeStruct((T, B), jnp.float32),
            jax.ShapeDtypeStruct((T - 1, B), jnp.float32),
        ],
        compiler_params=pltpu.CompilerParams(
            dimension_semantics=("parallel",),
        ),
    )(
        predicted_continue_logits.T,
        rewards.T,
        continues.T,
        critic_values.T,
    )
    return probs_t.T, returns_t.T


# speedup vs baseline: 18.7600x; 1.0179x over previous
import jax
import jax.numpy as jnp
from jax.experimental import pallas as pl
from jax.experimental.pallas import tpu as pltpu

_GAMMA = 0.997
_LAMDA = 0.95
_T = 64
_BLKB = 16384


def _shift_up(x, d, fill):
    pad = jnp.full((d, x.shape[1]), fill, dtype=x.dtype)
    return jnp.concatenate([x[d:, :], pad], axis=0)


def _loss_kernel(logits_ref, rewards_ref, continues_ref, values_ref,
                 probs_ref, returns_ref):
    probs_ref[...] = jax.nn.sigmoid(logits_ref[...])

    r = rewards_ref[...]
    c = continues_ref[...]
    v = values_ref[...]

    vn = _shift_up(v, 1, 0.0)
    a = r + (_GAMMA * (1.0 - _LAMDA)) * c * vn
    b = (_GAMMA * _LAMDA) * c

    row = jax.lax.broadcasted_iota(jnp.int32, a.shape, 0)
    last = row >= _T - 1
    a = jnp.where(last, 0.0, a)
    b = jnp.where(last, 1.0, b)

    for d in (1, 2, 4, 8, 16):
        a_s = _shift_up(a, d, 0.0)
        b_s = _shift_up(b, d, 1.0)
        a = a + b * a_s
        b = b * b_s

    bootstrap = v[_T - 1:_T, :]
    a_s = _shift_up(a, 32, 0.0)
    b_s = _shift_up(b, 32, 1.0)
    returns_ref[...] = (a + b * (a_s + b_s * bootstrap))[:_T - 1, :]


@jax.jit
def kernel(predicted_continue_logits, rewards, continues, critic_values):
    B, T = predicted_continue_logits.shape
    grid = (B // _BLKB,)
    in_spec = pl.BlockSpec((T, _BLKB), lambda i: (0, i))
    probs_t, returns_t = pl.pallas_call(
        _loss_kernel,
        grid=grid,
        in_specs=[in_spec, in_spec, in_spec, in_spec],
        out_specs=[
            pl.BlockSpec((T, _BLKB), lambda i: (0, i)),
            pl.BlockSpec((T - 1, _BLKB), lambda i: (0, i)),
        ],
        out_shape=[
            jax.ShapeDtypeStruct((T, B), jnp.float32),
            jax.ShapeDtypeStruct((T - 1, B), jnp.float32),
        ],
        compiler_params=pltpu.CompilerParams(
            dimension_semantics=("parallel",),
            vmem_limit_bytes=64 * 1024 * 1024,
        ),
    )(
        predicted_continue_logits.T,
        rewards.T,
        continues.T,
        critic_values.T,
    )
    return probs_t.T, returns_t.T
